# Initial kernel scaffold; baseline (speedup 1.0000x reference)
import jax
import jax.numpy as jnp
from jax.experimental import pallas as pl


def kernel(x_t, x_t_dt, edge_index, W_enc, b_enc, W_gcn, b_gcn, W_cls, b_cls):
    raise NotImplementedError("write your pallas kernel here")


# speedup vs baseline: 39.2305x; 39.2305x over previous
import functools

import jax
import jax.numpy as jnp
from jax import lax
from jax.experimental import pallas as pl
from jax.experimental.pallas import tpu as pltpu
from jax.experimental.pallas import tpu_sc as plsc

N = 100000
E = 1600000
NP = 102400
EP = 1638400
ER = EP // 128
HID = 32
HH = 16

NPT = NP // 16
BR = 4096
GRID = NP // BR


def _deg_body(ei3, zeros_hbm, out, didx, ones_v, deg_sp, sem):
    c = lax.axis_index("c")
    t = lax.axis_index("s")
    for k in range(8):
        ones_v[pl.ds(k * 16, 16)] = jnp.full((16,), 1.0, jnp.float32)
    pltpu.sync_copy(zeros_hbm.at[pl.ds(t * NPT, NPT)],
                    deg_sp.at[pl.ds(t * NPT, NPT)])
    plsc.subcore_barrier()

    w = c * 16 + t
    def chunk(i, carry):
        rb = w * 400 + i * 16
        pltpu.sync_copy(ei3.at[1, pl.ds(rb, 16)], didx)
        for j in range(16):
            pltpu.sync_copy(ones_v, deg_sp.at[didx.at[j]], add=True)
        return carry
    lax.fori_loop(0, 25, chunk, 0)
    plsc.subcore_barrier()
    pltpu.sync_copy(deg_sp.at[pl.ds(t * NPT, NPT)],
                    out.at[c, pl.ds(t * NPT, NPT)])


_deg_kernel = functools.partial(
    pl.kernel,
    mesh=plsc.VectorSubcoreMesh(core_axis_name="c", subcore_axis_name="s"),
    compiler_params=pltpu.CompilerParams(use_tc_tiling_on_sc=False),
    out_type=jax.ShapeDtypeStruct((2, NP), jnp.float32),
    scratch_types=[
        pltpu.VMEM((16, 128), jnp.int32),
        pltpu.VMEM((128,), jnp.float32),
        pltpu.VMEM_SHARED((NP,), jnp.float32),
        pltpu.SemaphoreType.DMA,
    ],
)(_deg_body)


def _agg_body(ei3, g0, g1, out, sidx, didx, rows, agg_sp, sem, sem2):
    c = lax.axis_index("c")
    t = lax.axis_index("s")

    def run(gt):
        pltpu.sync_copy(gt.at[pl.ds(t * NPT, NPT)],
                        agg_sp.at[pl.ds(t * NPT, NPT)])
        plsc.subcore_barrier()

        def chunk(i, carry):
            rb = t * 800 + i * 8
            pltpu.sync_copy(ei3.at[0, pl.ds(rb, 8)], sidx)
            pltpu.sync_copy(ei3.at[1, pl.ds(rb, 8)], didx)
            gcps = [
                pltpu.async_copy(gt.at[sidx.at[j]],
                                 rows.at[pl.ds(j * 128, 128)], sem)
                for j in range(8)
            ]
            scps = []
            for j in range(8):
                gcps[j].wait()
                scps.append(
                    pltpu.async_copy(rows.at[pl.ds(j * 128, 128)],
                                     agg_sp.at[didx.at[j]], sem2, add=True))
            for cp in scps:
                cp.wait()
            return carry
        lax.fori_loop(0, 100, chunk, 0)
        plsc.subcore_barrier()
        pltpu.sync_copy(agg_sp.at[pl.ds(t * NPT, NPT)],
                        out.at[c, pl.ds(t * NPT, NPT)])

    @pl.when(c == 0)
    def _():
        run(g0)

    @pl.when(c == 1)
    def _():
        run(g1)


_agg_kernel = functools.partial(
    pl.kernel,
    mesh=plsc.VectorSubcoreMesh(core_axis_name="c", subcore_axis_name="s"),
    compiler_params=pltpu.CompilerParams(use_tc_tiling_on_sc=False),
    out_type=jax.ShapeDtypeStruct((2, NP, HH), jnp.float32),
    scratch_types=[
        pltpu.VMEM((8, 128), jnp.int32),
        pltpu.VMEM((8, 128), jnp.int32),
        pltpu.VMEM((1024, HH), jnp.float32),
        pltpu.VMEM_SHARED((NP, HH), jnp.float32),
        pltpu.SemaphoreType.DMA,
        pltpu.SemaphoreType.DMA,
    ],
)(_agg_body)


def _score_body(ei3, sp_hbm, out, sidxb, didxb, stab, sbuf, sem):
    c = lax.axis_index("c")
    t = lax.axis_index("s")
    pltpu.sync_copy(sp_hbm, stab)
    himask = jnp.full((16,), -65536, jnp.int32)

    def chunk(i, carry):
        rb = c * 6400 + t * 400 + i * 16
        pltpu.sync_copy(ei3.at[0, pl.ds(rb, 16)], sidxb)
        pltpu.sync_copy(ei3.at[1, pl.ds(rb, 16)], didxb)

        def row(j, carry2):
            for k in range(8):
                sidx16 = sidxb[j, pl.ds(k * 16, 16)]
                didx16 = didxb[j, pl.ds(k * 16, 16)]
                ws = plsc.load_gather(stab, [sidx16])
                wd = plsc.load_gather(stab, [didx16])
                s1 = plsc.bitcast(ws & himask, jnp.float32)
                s2 = plsc.bitcast(lax.shift_left(wd, 16), jnp.float32)
                z = s1 + s2
                sbuf[j, pl.ds(k * 16, 16)] = 1.0 / (1.0 + jnp.exp(-z))
            return carry2
        lax.fori_loop(0, 16, row, 0)
        pltpu.sync_copy(sbuf, out.at[pl.ds(rb, 16)])
        return carry
    lax.fori_loop(0, 25, chunk, 0)


_score_kernel = functools.partial(
    pl.kernel,
    mesh=plsc.VectorSubcoreMesh(core_axis_name="c", subcore_axis_name="s"),
    compiler_params=pltpu.CompilerParams(use_tc_tiling_on_sc=False,
                                         needs_layout_passes=False),
    out_type=jax.ShapeDtypeStruct((ER, 128), jnp.float32),
    scratch_types=[
        pltpu.VMEM((16, 128), jnp.int32),
        pltpu.VMEM((16, 128), jnp.int32),
        pltpu.VMEM((NP,), jnp.int32),
        pltpu.VMEM((16, 128), jnp.float32),
        pltpu.SemaphoreType.DMA,
    ],
)(_score_body)


def _enc_body(xt_ref, wenc_ref, benc_ref, wgcn_ref, degp_ref, g0_ref, g1_ref):
    x = jnp.dot(xt_ref[...], wenc_ref[...], preferred_element_type=jnp.float32)
    x = jnp.maximum(x + benc_ref[...], 0.0)
    h = jnp.dot(x, wgcn_ref[...], preferred_element_type=jnp.float32)
    deg = degp_ref[0, :] + degp_ref[1, :] + 1.0
    dinv = lax.rsqrt(deg)
    g = h * dinv[:, None]
    g0_ref[...] = g[:, :HH]
    g1_ref[...] = g[:, HH:]


def _enc_call(xt_p, wenc_p, benc, wgcn, degp):
    return pl.pallas_call(
        _enc_body,
        grid=(GRID,),
        in_specs=[
            pl.BlockSpec((BR, 8), lambda i: (i, 0)),
            pl.BlockSpec((8, HID), lambda i: (0, 0)),
            pl.BlockSpec((1, HID), lambda i: (0, 0)),
            pl.BlockSpec((HID, HID), lambda i: (0, 0)),
            pl.BlockSpec((2, BR), lambda i: (0, i)),
        ],
        out_specs=[
            pl.BlockSpec((BR, HH), lambda i: (i, 0)),
            pl.BlockSpec((BR, HH), lambda i: (i, 0)),
        ],
        out_shape=[
            jax.ShapeDtypeStruct((NP, HH), jnp.float32),
            jax.ShapeDtypeStruct((NP, HH), jnp.float32),
        ],
    )(xt_p, wenc_p, benc, wgcn, degp)


def _round_bf16_hi(x):
    b = lax.bitcast_convert_type(x, jnp.int32)
    r = b + 0x7FFF + (lax.shift_right_logical(b, 16) & 1)
    return r & jnp.int32(-65536)


def _cls_body(agg_ref, degp_ref, bgcn_ref, wc1_ref, wc2_ref, bcls_ref, s_ref):
    ag = jnp.concatenate([agg_ref[0], agg_ref[1]], axis=1)
    deg = degp_ref[0, :] + degp_ref[1, :] + 1.0
    dinv = lax.rsqrt(deg)
    x2 = jnp.maximum(ag * dinv[:, None] + bgcn_ref[...], 0.0)
    s1 = jnp.sum(x2 * wc1_ref[...], axis=1) + bcls_ref[0, 0]
    s2 = jnp.sum(x2 * wc2_ref[...], axis=1)
    s_ref[0, :] = _round_bf16_hi(s1) | lax.shift_right_logical(
        _round_bf16_hi(s2), 16)


def _cls_call(agg, degp, bgcn, wc1, wc2, bcls):
    return pl.pallas_call(
        _cls_body,
        grid=(GRID,),
        in_specs=[
            pl.BlockSpec((2, BR, HH), lambda i: (0, i, 0)),
            pl.BlockSpec((2, BR), lambda i: (0, i)),
            pl.BlockSpec((1, HID), lambda i: (0, 0)),
            pl.BlockSpec((1, HID), lambda i: (0, 0)),
            pl.BlockSpec((1, HID), lambda i: (0, 0)),
            pl.BlockSpec((1, 1), lambda i: (0, 0)),
        ],
        out_specs=pl.BlockSpec((1, BR), lambda i: (0, i)),
        out_shape=jax.ShapeDtypeStruct((1, NP), jnp.int32),
    )(agg, degp, bgcn, wc1, wc2, bcls)


@jax.jit
def kernel(x_t, x_t_dt, edge_index, W_enc, b_enc, W_gcn, b_gcn, W_cls, b_cls):
    f32 = jnp.float32
    ei_p = jnp.full((2, EP), N, jnp.int32).at[:, :E].set(edge_index)
    ei3 = ei_p.reshape(2, ER, 128)

    xt_p = jnp.zeros((NP, 8), f32).at[:N, :7].set(x_t)
    wenc_p = jnp.zeros((8, HID), f32).at[:7, :].set(W_enc)
    benc = b_enc.reshape(1, HID)
    bgcn = b_gcn.reshape(1, HID)
    wc1 = W_cls[:HID, 0].reshape(1, HID)
    wc2 = W_cls[HID:, 0].reshape(1, HID)
    bcls = b_cls.reshape(1, 1)
    zeros_np = jnp.zeros((NP,), f32)

    degp = _deg_kernel(ei3, zeros_np)
    g0, g1 = _enc_call(xt_p, wenc_p, benc, W_gcn, degp)
    agg = _agg_kernel(ei3, g0, g1)
    s = _cls_call(agg, degp, bgcn, wc1, wc2, bcls)
    sco = _score_kernel(ei3, s.reshape(NP))
    return sco.reshape(EP)[:E]


# speedup vs baseline: 44.4518x; 1.1331x over previous
import functools

import jax
import jax.numpy as jnp
from jax import lax
from jax.experimental import pallas as pl
from jax.experimental.pallas import tpu as pltpu
from jax.experimental.pallas import tpu_sc as plsc

N = 100000
E = 1600000
NP = 102400
EP = 1638400
ER = EP // 128
HID = 32
HH = 16

NPT = NP // 16
BR = 4096
GRID = NP // BR


def _deg_body(ei3, zeros_hbm, out, didx, ones_v, deg_sp, sem):
    c = lax.axis_index("c")
    t = lax.axis_index("s")
    for k in range(8):
        ones_v[pl.ds(k * 16, 16)] = jnp.full((16,), 1.0, jnp.float32)
    pltpu.sync_copy(zeros_hbm.at[pl.ds(t * NPT, NPT)],
                    deg_sp.at[pl.ds(t * NPT, NPT)])
    plsc.subcore_barrier()

    w = c * 16 + t
    def chunk(i, carry):
        rb = w * 400 + i * 16
        pltpu.sync_copy(ei3.at[1, pl.ds(rb, 16)], didx)
        for j in range(16):
            pltpu.sync_copy(ones_v, deg_sp.at[didx.at[j]], add=True)
        return carry
    lax.fori_loop(0, 25, chunk, 0)
    plsc.subcore_barrier()
    pltpu.sync_copy(deg_sp.at[pl.ds(t * NPT, NPT)],
                    out.at[c, pl.ds(t * NPT, NPT)])


_deg_kernel = functools.partial(
    pl.kernel,
    mesh=plsc.VectorSubcoreMesh(core_axis_name="c", subcore_axis_name="s"),
    compiler_params=pltpu.CompilerParams(use_tc_tiling_on_sc=False),
    out_type=jax.ShapeDtypeStruct((2, NP), jnp.float32),
    scratch_types=[
        pltpu.VMEM((16, 128), jnp.int32),
        pltpu.VMEM((128,), jnp.float32),
        pltpu.VMEM_SHARED((NP,), jnp.float32),
        pltpu.SemaphoreType.DMA,
    ],
)(_deg_body)


AR = 4
ACH = 800 // AR
ARR = AR * 128


def _agg_body(ei3, g0, g1, out, sidx, didx, rows, agg_sp,
              si0, si1, si2, si3, sg0, sg1, ss0, ss1):
    c = lax.axis_index("c")
    t = lax.axis_index("s")
    sem_i = [si0, si1, si2, si3]
    sem_g = [sg0, sg1]
    sem_s = [ss0, ss1]

    def run(gt):
        def load_idx(cc, slot):
            rb = t * 800 + cc * AR
            pltpu.async_copy(ei3.at[0, pl.ds(rb, AR)],
                             sidx.at[pl.ds(slot * AR, AR)], sem_i[slot])
            pltpu.async_copy(ei3.at[1, pl.ds(rb, AR)],
                             didx.at[pl.ds(slot * AR, AR)], sem_i[slot])

        def wait_idx(slot):
            pltpu.make_async_copy(ei3.at[0, pl.ds(0, AR)],
                                  sidx.at[pl.ds(slot * AR, AR)],
                                  sem_i[slot]).wait()
            pltpu.make_async_copy(ei3.at[1, pl.ds(0, AR)],
                                  didx.at[pl.ds(slot * AR, AR)],
                                  sem_i[slot]).wait()

        def issue_gathers(slot, buf):
            for j in range(AR):
                pltpu.async_copy(gt.at[sidx.at[slot * AR + j]],
                                 rows.at[pl.ds(buf * ARR + j * 128, 128)],
                                 sem_g[buf])

        def wait_rows(sem, buf):
            pltpu.make_async_copy(gt.at[pl.ds(0, ARR)],
                                  rows.at[pl.ds(buf * ARR, ARR)],
                                  sem[buf]).wait()

        def issue_scatters(slot, buf):
            for j in range(AR):
                pltpu.async_copy(rows.at[pl.ds(buf * ARR + j * 128, 128)],
                                 agg_sp.at[didx.at[slot * AR + j]],
                                 sem_s[buf], add=True)

        for s in range(4):
            load_idx(s, s)
        pltpu.sync_copy(gt.at[pl.ds(t * NPT, NPT)],
                        agg_sp.at[pl.ds(t * NPT, NPT)])
        plsc.subcore_barrier()

        for b in range(4):
            if b >= 2:
                wait_rows(sem_s, b % 2)
                load_idx(b + 2, (b + 2) % 4)
            wait_idx(b)
            issue_gathers(b, b % 2)
            if b >= 1:
                wait_rows(sem_g, (b - 1) % 2)
                issue_scatters(b - 1, (b - 1) % 2)

        def body(k, carry):
            for b in range(4):
                cc = 4 * k + b
                wait_rows(sem_s, b % 2)
                rb2 = t * 800 + lax.rem(cc + 2, ACH) * AR
                pltpu.async_copy(ei3.at[0, pl.ds(rb2, AR)],
                                 sidx.at[pl.ds(((b + 2) % 4) * AR, AR)],
                                 sem_i[(b + 2) % 4])
                pltpu.async_copy(ei3.at[1, pl.ds(rb2, AR)],
                                 didx.at[pl.ds(((b + 2) % 4) * AR, AR)],
                                 sem_i[(b + 2) % 4])
                wait_idx(b)
                issue_gathers(b, b % 2)
                wait_rows(sem_g, (b - 1) % 2)
                issue_scatters((b - 1) % 4, (b - 1) % 2)
            return carry
        lax.fori_loop(1, 50, body, 0)

        wait_rows(sem_g, 1)
        issue_scatters(3, 1)
        wait_rows(sem_s, 0)
        wait_rows(sem_s, 1)
        wait_idx(0)
        wait_idx(1)
        plsc.subcore_barrier()
        pltpu.sync_copy(agg_sp.at[pl.ds(t * NPT, NPT)],
                        out.at[c, pl.ds(t * NPT, NPT)])

    @pl.when(c == 0)
    def _():
        run(g0)

    @pl.when(c == 1)
    def _():
        run(g1)


_agg_kernel = functools.partial(
    pl.kernel,
    mesh=plsc.VectorSubcoreMesh(core_axis_name="c", subcore_axis_name="s"),
    compiler_params=pltpu.CompilerParams(use_tc_tiling_on_sc=False),
    out_type=jax.ShapeDtypeStruct((2, NP, HH), jnp.float32),
    scratch_types=[
        pltpu.VMEM((16, 128), jnp.int32),
        pltpu.VMEM((16, 128), jnp.int32),
        pltpu.VMEM((2 * ARR, HH), jnp.float32),
        pltpu.VMEM_SHARED((NP, HH), jnp.float32),
        pltpu.SemaphoreType.DMA,
        pltpu.SemaphoreType.DMA,
        pltpu.SemaphoreType.DMA,
        pltpu.SemaphoreType.DMA,
        pltpu.SemaphoreType.DMA,
        pltpu.SemaphoreType.DMA,
        pltpu.SemaphoreType.DMA,
        pltpu.SemaphoreType.DMA,
    ],
)(_agg_body)


def _score_body(ei3, sp_hbm, out, sidxb, didxb, stab, sbuf, sem):
    c = lax.axis_index("c")
    t = lax.axis_index("s")
    pltpu.sync_copy(sp_hbm, stab)
    himask = jnp.full((16,), -65536, jnp.int32)

    def chunk(i, carry):
        rb = c * 6400 + t * 400 + i * 16
        pltpu.sync_copy(ei3.at[0, pl.ds(rb, 16)], sidxb)
        pltpu.sync_copy(ei3.at[1, pl.ds(rb, 16)], didxb)

        def row(j, carry2):
            for k in range(8):
                sidx16 = sidxb[j, pl.ds(k * 16, 16)]
                didx16 = didxb[j, pl.ds(k * 16, 16)]
                ws = plsc.load_gather(stab, [sidx16])
                wd = plsc.load_gather(stab, [didx16])
                s1 = plsc.bitcast(ws & himask, jnp.float32)
                s2 = plsc.bitcast(lax.shift_left(wd, 16), jnp.float32)
                z = s1 + s2
                sbuf[j, pl.ds(k * 16, 16)] = 1.0 / (1.0 + jnp.exp(-z))
            return carry2
        lax.fori_loop(0, 16, row, 0)
        pltpu.sync_copy(sbuf, out.at[pl.ds(rb, 16)])
        return carry
    lax.fori_loop(0, 25, chunk, 0)


_score_kernel = functools.partial(
    pl.kernel,
    mesh=plsc.VectorSubcoreMesh(core_axis_name="c", subcore_axis_name="s"),
    compiler_params=pltpu.CompilerParams(use_tc_tiling_on_sc=False,
                                         needs_layout_passes=False),
    out_type=jax.ShapeDtypeStruct((ER, 128), jnp.float32),
    scratch_types=[
        pltpu.VMEM((16, 128), jnp.int32),
        pltpu.VMEM((16, 128), jnp.int32),
        pltpu.VMEM((NP,), jnp.int32),
        pltpu.VMEM((16, 128), jnp.float32),
        pltpu.SemaphoreType.DMA,
    ],
)(_score_body)


def _enc_body(xt_ref, wenc_ref, benc_ref, wgcn_ref, degp_ref, g0_ref, g1_ref):
    x = jnp.dot(xt_ref[...], wenc_ref[...], preferred_element_type=jnp.float32)
    x = jnp.maximum(x + benc_ref[...], 0.0)
    h = jnp.dot(x, wgcn_ref[...], preferred_element_type=jnp.float32)
    deg = degp_ref[0, :] + degp_ref[1, :] + 1.0
    dinv = lax.rsqrt(deg)
    g = h * dinv[:, None]
    g0_ref[...] = g[:, :HH]
    g1_ref[...] = g[:, HH:]


def _enc_call(xt_p, wenc_p, benc, wgcn, degp):
    return pl.pallas_call(
        _enc_body,
        grid=(GRID,),
        in_specs=[
            pl.BlockSpec((BR, 8), lambda i: (i, 0)),
            pl.BlockSpec((8, HID), lambda i: (0, 0)),
            pl.BlockSpec((1, HID), lambda i: (0, 0)),
            pl.BlockSpec((HID, HID), lambda i: (0, 0)),
            pl.BlockSpec((2, BR), lambda i: (0, i)),
        ],
        out_specs=[
            pl.BlockSpec((BR, HH), lambda i: (i, 0)),
            pl.BlockSpec((BR, HH), lambda i: (i, 0)),
        ],
        out_shape=[
            jax.ShapeDtypeStruct((NP, HH), jnp.float32),
            jax.ShapeDtypeStruct((NP, HH), jnp.float32),
        ],
    )(xt_p, wenc_p, benc, wgcn, degp)


def _round_bf16_hi(x):
    b = lax.bitcast_convert_type(x, jnp.int32)
    r = b + 0x7FFF + (lax.shift_right_logical(b, 16) & 1)
    return r & jnp.int32(-65536)


def _cls_body(agg_ref, degp_ref, bgcn_ref, wc1_ref, wc2_ref, bcls_ref, s_ref):
    ag = jnp.concatenate([agg_ref[0], agg_ref[1]], axis=1)
    deg = degp_ref[0, :] + degp_ref[1, :] + 1.0
    dinv = lax.rsqrt(deg)
    x2 = jnp.maximum(ag * dinv[:, None] + bgcn_ref[...], 0.0)
    s1 = jnp.sum(x2 * wc1_ref[...], axis=1) + bcls_ref[0, 0]
    s2 = jnp.sum(x2 * wc2_ref[...], axis=1)
    s_ref[0, :] = _round_bf16_hi(s1) | lax.shift_right_logical(
        _round_bf16_hi(s2), 16)


def _cls_call(agg, degp, bgcn, wc1, wc2, bcls):
    return pl.pallas_call(
        _cls_body,
        grid=(GRID,),
        in_specs=[
            pl.BlockSpec((2, BR, HH), lambda i: (0, i, 0)),
            pl.BlockSpec((2, BR), lambda i: (0, i)),
            pl.BlockSpec((1, HID), lambda i: (0, 0)),
            pl.BlockSpec((1, HID), lambda i: (0, 0)),
            pl.BlockSpec((1, HID), lambda i: (0, 0)),
            pl.BlockSpec((1, 1), lambda i: (0, 0)),
        ],
        out_specs=pl.BlockSpec((1, BR), lambda i: (0, i)),
        out_shape=jax.ShapeDtypeStruct((1, NP), jnp.int32),
    )(agg, degp, bgcn, wc1, wc2, bcls)


@jax.jit
def kernel(x_t, x_t_dt, edge_index, W_enc, b_enc, W_gcn, b_gcn, W_cls, b_cls):
    f32 = jnp.float32
    ei_p = jnp.full((2, EP), N, jnp.int32).at[:, :E].set(edge_index)
    ei3 = ei_p.reshape(2, ER, 128)

    xt_p = jnp.zeros((NP, 8), f32).at[:N, :7].set(x_t)
    wenc_p = jnp.zeros((8, HID), f32).at[:7, :].set(W_enc)
    benc = b_enc.reshape(1, HID)
    bgcn = b_gcn.reshape(1, HID)
    wc1 = W_cls[:HID, 0].reshape(1, HID)
    wc2 = W_cls[HID:, 0].reshape(1, HID)
    bcls = b_cls.reshape(1, 1)
    zeros_np = jnp.zeros((NP,), f32)

    degp = _deg_kernel(ei3, zeros_np)
    g0, g1 = _enc_call(xt_p, wenc_p, benc, W_gcn, degp)
    agg = _agg_kernel(ei3, g0, g1)
    s = _cls_call(agg, degp, bgcn, wc1, wc2, bcls)
    sco = _score_kernel(ei3, s.reshape(NP))
    return sco.reshape(EP)[:E]


# speedup vs baseline: 44.7411x; 1.0065x over previous
import functools

import jax
import jax.numpy as jnp
from jax import lax
from jax.experimental import pallas as pl
from jax.experimental.pallas import tpu as pltpu
from jax.experimental.pallas import tpu_sc as plsc

N = 100000
E = 1600000
NP = 102400
EP = 1638400
ER = EP // 128
HID = 32
HH = 16

NPT = NP // 16
BR = 12800
GRID = NP // BR


def _deg_body(ei3, zeros_hbm, out, didx, ones_v, deg_sp, sem):
    c = lax.axis_index("c")
    t = lax.axis_index("s")
    for k in range(8):
        ones_v[pl.ds(k * 16, 16)] = jnp.full((16,), 1.0, jnp.float32)
    pltpu.sync_copy(zeros_hbm.at[pl.ds(t * NPT, NPT)],
                    deg_sp.at[pl.ds(t * NPT, NPT)])
    plsc.subcore_barrier()

    w = c * 16 + t
    def chunk(i, carry):
        rb = w * 400 + i * 16
        pltpu.sync_copy(ei3.at[1, pl.ds(rb, 16)], didx)
        for j in range(16):
            pltpu.sync_copy(ones_v, deg_sp.at[didx.at[j]], add=True)
        return carry
    lax.fori_loop(0, 25, chunk, 0)
    plsc.subcore_barrier()
    pltpu.sync_copy(deg_sp.at[pl.ds(t * NPT, NPT)],
                    out.at[c, pl.ds(t * NPT, NPT)])


_deg_kernel = functools.partial(
    pl.kernel,
    mesh=plsc.VectorSubcoreMesh(core_axis_name="c", subcore_axis_name="s"),
    compiler_params=pltpu.CompilerParams(use_tc_tiling_on_sc=False),
    out_type=jax.ShapeDtypeStruct((2, NP), jnp.float32),
    scratch_types=[
        pltpu.VMEM((16, 128), jnp.int32),
        pltpu.VMEM((128,), jnp.float32),
        pltpu.VMEM_SHARED((NP,), jnp.float32),
        pltpu.SemaphoreType.DMA,
    ],
)(_deg_body)


AR = 4
ACH = 800 // AR
ARR = AR * 128


def _agg_body(ei3, g0, g1, out, sidx, didx, rows, agg_sp,
              si0, si1, si2, si3, sg0, sg1, ss0, ss1):
    c = lax.axis_index("c")
    t = lax.axis_index("s")
    sem_i = [si0, si1, si2, si3]
    sem_g = [sg0, sg1]
    sem_s = [ss0, ss1]

    def run(gt):
        def load_idx(cc, slot):
            rb = t * 800 + cc * AR
            pltpu.async_copy(ei3.at[0, pl.ds(rb, AR)],
                             sidx.at[pl.ds(slot * AR, AR)], sem_i[slot])
            pltpu.async_copy(ei3.at[1, pl.ds(rb, AR)],
                             didx.at[pl.ds(slot * AR, AR)], sem_i[slot])

        def wait_idx(slot):
            pltpu.make_async_copy(ei3.at[0, pl.ds(0, AR)],
                                  sidx.at[pl.ds(slot * AR, AR)],
                                  sem_i[slot]).wait()
            pltpu.make_async_copy(ei3.at[1, pl.ds(0, AR)],
                                  didx.at[pl.ds(slot * AR, AR)],
                                  sem_i[slot]).wait()

        def issue_gathers(slot, buf):
            for j in range(AR):
                pltpu.async_copy(gt.at[sidx.at[slot * AR + j]],
                                 rows.at[pl.ds(buf * ARR + j * 128, 128)],
                                 sem_g[buf])

        def wait_rows(sem, buf):
            pltpu.make_async_copy(gt.at[pl.ds(0, ARR)],
                                  rows.at[pl.ds(buf * ARR, ARR)],
                                  sem[buf]).wait()

        def issue_scatters(slot, buf):
            for j in range(AR):
                pltpu.async_copy(rows.at[pl.ds(buf * ARR + j * 128, 128)],
                                 agg_sp.at[didx.at[slot * AR + j]],
                                 sem_s[buf], add=True)

        for s in range(4):
            load_idx(s, s)
        pltpu.sync_copy(gt.at[pl.ds(t * NPT, NPT)],
                        agg_sp.at[pl.ds(t * NPT, NPT)])
        plsc.subcore_barrier()

        for b in range(4):
            if b >= 2:
                wait_rows(sem_s, b % 2)
                load_idx(b + 2, (b + 2) % 4)
            wait_idx(b)
            issue_gathers(b, b % 2)
            if b >= 1:
                wait_rows(sem_g, (b - 1) % 2)
                issue_scatters(b - 1, (b - 1) % 2)

        def body(k, carry):
            for b in range(4):
                cc = 4 * k + b
                wait_rows(sem_s, b % 2)
                rb2 = t * 800 + lax.rem(cc + 2, ACH) * AR
                pltpu.async_copy(ei3.at[0, pl.ds(rb2, AR)],
                                 sidx.at[pl.ds(((b + 2) % 4) * AR, AR)],
                                 sem_i[(b + 2) % 4])
                pltpu.async_copy(ei3.at[1, pl.ds(rb2, AR)],
                                 didx.at[pl.ds(((b + 2) % 4) * AR, AR)],
                                 sem_i[(b + 2) % 4])
                wait_idx(b)
                issue_gathers(b, b % 2)
                wait_rows(sem_g, (b - 1) % 2)
                issue_scatters((b - 1) % 4, (b - 1) % 2)
            return carry
        lax.fori_loop(1, 50, body, 0)

        wait_rows(sem_g, 1)
        issue_scatters(3, 1)
        wait_rows(sem_s, 0)
        wait_rows(sem_s, 1)
        wait_idx(0)
        wait_idx(1)
        plsc.subcore_barrier()
        pltpu.sync_copy(agg_sp.at[pl.ds(t * NPT, NPT)],
                        out.at[c, pl.ds(t * NPT, NPT)])

    @pl.when(c == 0)
    def _():
        run(g0)

    @pl.when(c == 1)
    def _():
        run(g1)


_agg_kernel = functools.partial(
    pl.kernel,
    mesh=plsc.VectorSubcoreMesh(core_axis_name="c", subcore_axis_name="s"),
    compiler_params=pltpu.CompilerParams(use_tc_tiling_on_sc=False),
    out_type=jax.ShapeDtypeStruct((2, NP, HH), jnp.float32),
    scratch_types=[
        pltpu.VMEM((16, 128), jnp.int32),
        pltpu.VMEM((16, 128), jnp.int32),
        pltpu.VMEM((2 * ARR, HH), jnp.float32),
        pltpu.VMEM_SHARED((NP, HH), jnp.float32),
        pltpu.SemaphoreType.DMA,
        pltpu.SemaphoreType.DMA,
        pltpu.SemaphoreType.DMA,
        pltpu.SemaphoreType.DMA,
        pltpu.SemaphoreType.DMA,
        pltpu.SemaphoreType.DMA,
        pltpu.SemaphoreType.DMA,
        pltpu.SemaphoreType.DMA,
    ],
)(_agg_body)


def _score_body(ei3, sp_hbm, out, sidxb, didxb, stab, sbuf, sem):
    c = lax.axis_index("c")
    t = lax.axis_index("s")
    pltpu.sync_copy(sp_hbm, stab)
    himask = jnp.full((16,), -65536, jnp.int32)

    def chunk(i, carry):
        rb = c * 6400 + t * 400 + i * 16
        pltpu.sync_copy(ei3.at[0, pl.ds(rb, 16)], sidxb)
        pltpu.sync_copy(ei3.at[1, pl.ds(rb, 16)], didxb)

        def row(j, carry2):
            for k in range(8):
                sidx16 = sidxb[j, pl.ds(k * 16, 16)]
                didx16 = didxb[j, pl.ds(k * 16, 16)]
                ws = plsc.load_gather(stab, [sidx16])
                wd = plsc.load_gather(stab, [didx16])
                s1 = plsc.bitcast(ws & himask, jnp.float32)
                s2 = plsc.bitcast(lax.shift_left(wd, 16), jnp.float32)
                z = s1 + s2
                sbuf[j, pl.ds(k * 16, 16)] = 1.0 / (1.0 + jnp.exp(-z))
            return carry2
        lax.fori_loop(0, 16, row, 0)
        pltpu.sync_copy(sbuf, out.at[pl.ds(rb, 16)])
        return carry
    lax.fori_loop(0, 25, chunk, 0)


_score_kernel = functools.partial(
    pl.kernel,
    mesh=plsc.VectorSubcoreMesh(core_axis_name="c", subcore_axis_name="s"),
    compiler_params=pltpu.CompilerParams(use_tc_tiling_on_sc=False,
                                         needs_layout_passes=False),
    out_type=jax.ShapeDtypeStruct((ER, 128), jnp.float32),
    scratch_types=[
        pltpu.VMEM((16, 128), jnp.int32),
        pltpu.VMEM((16, 128), jnp.int32),
        pltpu.VMEM((NP,), jnp.int32),
        pltpu.VMEM((16, 128), jnp.float32),
        pltpu.SemaphoreType.DMA,
    ],
)(_score_body)


def _enc_body(xt_ref, wenc_ref, benc_ref, wgcn_ref, degp_ref, g0_ref, g1_ref):
    x = jnp.dot(xt_ref[...], wenc_ref[...], preferred_element_type=jnp.float32)
    x = jnp.maximum(x + benc_ref[...], 0.0)
    h = jnp.dot(x, wgcn_ref[...], preferred_element_type=jnp.float32)
    deg = degp_ref[0, :] + degp_ref[1, :] + 1.0
    dinv = lax.rsqrt(deg)
    g = h * dinv[:, None]
    g0_ref[...] = g[:, :HH]
    g1_ref[...] = g[:, HH:]


def _enc_call(xt_p, wenc_p, benc, wgcn, degp):
    return pl.pallas_call(
        _enc_body,
        grid=(GRID,),
        in_specs=[
            pl.BlockSpec((BR, 8), lambda i: (i, 0)),
            pl.BlockSpec((8, HID), lambda i: (0, 0)),
            pl.BlockSpec((1, HID), lambda i: (0, 0)),
            pl.BlockSpec((HID, HID), lambda i: (0, 0)),
            pl.BlockSpec((2, BR), lambda i: (0, i)),
        ],
        out_specs=[
            pl.BlockSpec((BR, HH), lambda i: (i, 0)),
            pl.BlockSpec((BR, HH), lambda i: (i, 0)),
        ],
        out_shape=[
            jax.ShapeDtypeStruct((NP, HH), jnp.float32),
            jax.ShapeDtypeStruct((NP, HH), jnp.float32),
        ],
    )(xt_p, wenc_p, benc, wgcn, degp)


def _round_bf16_hi(x):
    b = lax.bitcast_convert_type(x, jnp.int32)
    r = b + 0x7FFF + (lax.shift_right_logical(b, 16) & 1)
    return r & jnp.int32(-65536)


def _cls_body(agg_ref, degp_ref, bgcn_ref, wc1_ref, wc2_ref, bcls_ref, s_ref):
    ag = jnp.concatenate([agg_ref[0], agg_ref[1]], axis=1)
    deg = degp_ref[0, :] + degp_ref[1, :] + 1.0
    dinv = lax.rsqrt(deg)
    x2 = jnp.maximum(ag * dinv[:, None] + bgcn_ref[...], 0.0)
    s1 = jnp.sum(x2 * wc1_ref[...], axis=1) + bcls_ref[0, 0]
    s2 = jnp.sum(x2 * wc2_ref[...], axis=1)
    s_ref[0, :] = _round_bf16_hi(s1) | lax.shift_right_logical(
        _round_bf16_hi(s2), 16)


def _cls_call(agg, degp, bgcn, wc1, wc2, bcls):
    return pl.pallas_call(
        _cls_body,
        grid=(GRID,),
        in_specs=[
            pl.BlockSpec((2, BR, HH), lambda i: (0, i, 0)),
            pl.BlockSpec((2, BR), lambda i: (0, i)),
            pl.BlockSpec((1, HID), lambda i: (0, 0)),
            pl.BlockSpec((1, HID), lambda i: (0, 0)),
            pl.BlockSpec((1, HID), lambda i: (0, 0)),
            pl.BlockSpec((1, 1), lambda i: (0, 0)),
        ],
        out_specs=pl.BlockSpec((1, BR), lambda i: (0, i)),
        out_shape=jax.ShapeDtypeStruct((1, NP), jnp.int32),
    )(agg, degp, bgcn, wc1, wc2, bcls)


@jax.jit
def kernel(x_t, x_t_dt, edge_index, W_enc, b_enc, W_gcn, b_gcn, W_cls, b_cls):
    f32 = jnp.float32
    ei_p = jnp.full((2, EP), N, jnp.int32).at[:, :E].set(edge_index)
    ei3 = ei_p.reshape(2, ER, 128)

    xt_p = jnp.zeros((NP, 8), f32).at[:N, :7].set(x_t)
    wenc_p = jnp.zeros((8, HID), f32).at[:7, :].set(W_enc)
    benc = b_enc.reshape(1, HID)
    bgcn = b_gcn.reshape(1, HID)
    wc1 = W_cls[:HID, 0].reshape(1, HID)
    wc2 = W_cls[HID:, 0].reshape(1, HID)
    bcls = b_cls.reshape(1, 1)
    zeros_np = jnp.zeros((NP,), f32)

    degp = _deg_kernel(ei3, zeros_np)
    g0, g1 = _enc_call(xt_p, wenc_p, benc, W_gcn, degp)
    agg = _agg_kernel(ei3, g0, g1)
    s = _cls_call(agg, degp, bgcn, wc1, wc2, bcls)
    sco = _score_kernel(ei3, s.reshape(NP))
    return sco.reshape(EP)[:E]
